# Initial kernel scaffold; baseline (speedup 1.0000x reference)
import jax
import jax.numpy as jnp
from jax.experimental import pallas as pl


def kernel(input_ids, token_type_ids, item_position_ids, word_embeddings, position_embeddings, token_type_embeddings, item_position_embeddings, ln_gamma, ln_beta):
    raise NotImplementedError("write your pallas kernel here")


# speedup vs baseline: 5.6729x; 5.6729x over previous
import jax
import jax.numpy as jnp
from jax import lax
from jax.experimental import pallas as pl
from jax.experimental.pallas import tpu as pltpu
from jax.experimental.pallas import tpu_sc as plsc

VOCAB = 100000
HIDDEN = 128
PAD_IDX = 1
B, L = 4096, 200
EPS = 1e-5
LPAD = 208
NG = LPAD // 16
POS_ROWS = 208
NH = HIDDEN // 16


def _rsqrt_vec(v):
    i = plsc.bitcast(v, jnp.int32)
    y = plsc.bitcast(jnp.int32(0x5F3759DF) - (i >> 1), jnp.float32)
    for _ in range(3):
        y = y * (1.5 - 0.5 * v * y * y)
    return y


def _body(nc, ids_hbm, tt_hbm, ip_hbm, we_hbm, pe_hbm, tte_hbm, ipe_hbm,
          g_hbm, b_hbm, out_hbm,
          pe_v, c_v, tte_v, ipe_v, g_v, b_v,
          ids_v, tt_v, ip_v, pos_v, comb_v, acc, sem):
    rows_per_w = B // (nc * 16)
    wid = lax.axis_index("s") * nc + lax.axis_index("c")

    pltpu.sync_copy(pe_hbm.at[pl.ds(0, POS_ROWS)], pe_v)
    pltpu.sync_copy(tte_hbm, tte_v)
    pltpu.sync_copy(ipe_hbm, ipe_v)
    pltpu.sync_copy(g_hbm, g_v)
    pltpu.sync_copy(b_hbm, b_v)

    def build_c(r, carry):
        t = r >> 5
        ii = r & 31
        for j in range(NH):
            sj = pl.ds(16 * j, 16)
            c_v[r, sj] = tte_v[t, sj] + ipe_v[ii, sj]
        return carry
    lax.fori_loop(0, 128, build_c, 0)

    gvec = [g_v[pl.ds(16 * j, 16)] for j in range(NH)]
    bvec = [b_v[pl.ds(16 * j, 16)] for j in range(NH)]

    def row_body(i, carry0):
        row = wid * rows_per_w + i
        pltpu.sync_copy(ids_hbm.at[row], ids_v.at[pl.ds(0, L)])
        pltpu.sync_copy(tt_hbm.at[row], tt_v.at[pl.ds(0, L)])
        pltpu.sync_copy(ip_hbm.at[row], ip_v.at[pl.ds(0, L)])

        carry = jnp.float32(0.0)
        for g in range(NG):
            sl = pl.ds(16 * g, 16)
            idsg = ids_v[sl]
            m = jnp.where(idsg != PAD_IDX, jnp.float32(1.0), jnp.float32(0.0))
            cs = plsc.cumsum(m) + carry
            carry = carry + jnp.sum(m)
            pos = (cs * m).astype(jnp.int32) + PAD_IDX
            cmb = tt_v[sl] * 32 + ip_v[sl]
            if 16 * (g + 1) > L:
                valid = lax.iota(jnp.int32, 16) < (L - 16 * g)
                pos = jnp.where(valid, pos, 0)
                cmb = jnp.where(valid, cmb, 0)
            pos_v[sl] = pos
            comb_v[sl] = cmb

        cp1 = pltpu.async_copy(we_hbm.at[ids_v.at[pl.ds(0, 104)]],
                               acc.at[pl.ds(0, 104)], sem)
        cp2 = pltpu.async_copy(we_hbm.at[ids_v.at[pl.ds(104, 96)]],
                               acc.at[pl.ds(104, 96)], sem)
        cp1.wait()
        cp2.wait()

        def grp_body(g, carry1):
            tbase = 16 * g
            pv = pos_v[pl.ds(tbase, 16)]
            cv = comb_v[pl.ds(tbase, 16)]
            for k in range(16):
                p = pv[k]
                cmb = cv[k]
                t = tbase + k
                x = []
                for j in range(NH):
                    sj = pl.ds(16 * j, 16)
                    x.append(acc[t, sj] + pe_v[p, sj] + c_v[cmb, sj])
                s = x[0]
                ssq = x[0] * x[0]
                for j in range(1, NH):
                    s = s + x[j]
                    ssq = ssq + x[j] * x[j]
                mean = jnp.sum(s) * (1.0 / HIDDEN)
                var = jnp.sum(ssq) * (1.0 / HIDDEN) - mean * mean + EPS
                vs = _rsqrt_vec(jnp.broadcast_to(var, (16,)))
                meanv = jnp.broadcast_to(mean, (16,))
                for j in range(NH):
                    g2 = vs * gvec[j]
                    b2 = bvec[j] - meanv * g2
                    acc[t, pl.ds(16 * j, 16)] = x[j] * g2 + b2
            return carry1
        lax.fori_loop(0, NG, grp_body, 0)

        pltpu.sync_copy(acc.at[pl.ds(0, L)], out_hbm.at[row])
        return carry0
    lax.fori_loop(0, rows_per_w, row_body, 0)


def kernel(input_ids, token_type_ids, item_position_ids, word_embeddings,
           position_embeddings, token_type_embeddings, item_position_embeddings,
           ln_gamma, ln_beta):
    ids = input_ids.astype(jnp.int32)
    tt = token_type_ids.astype(jnp.int32)
    ip = item_position_ids.astype(jnp.int32)

    info = plsc.get_sparse_core_info()
    nc = info.num_cores
    mesh = plsc.VectorSubcoreMesh(core_axis_name="c", subcore_axis_name="s")
    scratch = [
        pltpu.VMEM((POS_ROWS, HIDDEN), jnp.float32),
        pltpu.VMEM((128, HIDDEN), jnp.float32),
        pltpu.VMEM((4, HIDDEN), jnp.float32),
        pltpu.VMEM((32, HIDDEN), jnp.float32),
        pltpu.VMEM((HIDDEN,), jnp.float32),
        pltpu.VMEM((HIDDEN,), jnp.float32),
        pltpu.VMEM((LPAD,), jnp.int32),
        pltpu.VMEM((LPAD,), jnp.int32),
        pltpu.VMEM((LPAD,), jnp.int32),
        pltpu.VMEM((LPAD,), jnp.int32),
        pltpu.VMEM((LPAD,), jnp.int32),
        pltpu.VMEM((LPAD, HIDDEN), jnp.float32),
        pltpu.SemaphoreType.DMA,
    ]

    def body(*refs):
        _body(nc, *refs)

    f = pl.kernel(
        body,
        out_type=jax.ShapeDtypeStruct((B, L, HIDDEN), jnp.float32),
        mesh=mesh,
        scratch_types=scratch,
        compiler_params=pltpu.CompilerParams(
            needs_layout_passes=False, use_tc_tiling_on_sc=False),
    )
    return f(ids, tt, ip, word_embeddings, position_embeddings,
             token_type_embeddings, item_position_embeddings,
             ln_gamma, ln_beta)


# speedup vs baseline: 6.5654x; 1.1573x over previous
import jax
import jax.numpy as jnp
from jax import lax
from jax.experimental import pallas as pl
from jax.experimental.pallas import tpu as pltpu
from jax.experimental.pallas import tpu_sc as plsc

VOCAB = 100000
HIDDEN = 128
PAD_IDX = 1
B, L = 4096, 200
EPS = 1e-5
LPAD = 208
NG = LPAD // 16
POS_ROWS = 208
NH = HIDDEN // 16
C0, C1 = 104, 96


def _rsqrt_vec(v):
    i = plsc.bitcast(v, jnp.int32)
    y = plsc.bitcast(jnp.int32(0x5F3759DF) - (i >> 1), jnp.float32)
    for _ in range(3):
        y = y * (1.5 - 0.5 * v * y * y)
    return y


def _body(nc, ids_hbm, tt_hbm, ip_hbm, we_hbm, pe_hbm, tte_hbm, ipe_hbm,
          g_hbm, b_hbm, out_hbm,
          pe_v, c_v, tte_v, ipe_v, g_v, b_v,
          ids0, tt0, ip0, pos0, comb0, acc0,
          ids1, tt1, ip1, pos1, comb1, acc1,
          sem_i0, sem_i1, sem_g0, sem_g1, sem_o0, sem_o1):
    rows_per_w = B // (nc * 16)
    wid = lax.axis_index("s") * nc + lax.axis_index("c")
    base = wid * rows_per_w
    last = rows_per_w - 1

    idsb = (ids0, ids1)
    ttb = (tt0, tt1)
    ipb = (ip0, ip1)
    posb = (pos0, pos1)
    combb = (comb0, comb1)
    accb = (acc0, acc1)
    sem_i = (sem_i0, sem_i1)
    sem_g = (sem_g0, sem_g1)
    sem_o = (sem_o0, sem_o1)

    pltpu.sync_copy(pe_hbm.at[pl.ds(0, POS_ROWS)], pe_v)
    pltpu.sync_copy(tte_hbm, tte_v)
    pltpu.sync_copy(ipe_hbm, ipe_v)
    pltpu.sync_copy(g_hbm, g_v)
    pltpu.sync_copy(b_hbm, b_v)

    def build_c(r, carry):
        t = r >> 5
        ii = r & 31
        for j in range(NH):
            sj = pl.ds(16 * j, 16)
            c_v[r, sj] = tte_v[t, sj] + ipe_v[ii, sj]
        return carry
    lax.fori_loop(0, 128, build_c, 0)

    gvec = [g_v[pl.ds(16 * j, 16)] for j in range(NH)]
    bvec = [b_v[pl.ds(16 * j, 16)] for j in range(NH)]

    def prefetch_ids(p, r):
        row = base + r
        return (
            pltpu.async_copy(ids_hbm.at[row], idsb[p].at[pl.ds(0, L)], sem_i[p]),
            pltpu.async_copy(tt_hbm.at[row], ttb[p].at[pl.ds(0, L)], sem_i[p]),
            pltpu.async_copy(ip_hbm.at[row], ipb[p].at[pl.ds(0, L)], sem_i[p]),
        )

    def start_gather(p):
        h1 = pltpu.async_copy(we_hbm.at[idsb[p].at[pl.ds(0, C0)]],
                              accb[p].at[pl.ds(0, C0)], sem_g[p])
        h2 = pltpu.async_copy(we_hbm.at[idsb[p].at[pl.ds(C0, C1)]],
                              accb[p].at[pl.ds(C0, C1)], sem_g[p])
        return h1, h2

    def start_out(p, r):
        return pltpu.async_copy(accb[p].at[pl.ds(0, L)], out_hbm.at[base + r],
                                sem_o[p])

    def cumsum_comb(p):
        carry = jnp.float32(0.0)
        for g in range(NG):
            sl = pl.ds(16 * g, 16)
            idsg = idsb[p][sl]
            m = jnp.where(idsg != PAD_IDX, jnp.float32(1.0), jnp.float32(0.0))
            cs = plsc.cumsum(m) + carry
            carry = carry + jnp.sum(m)
            pos = (cs * m).astype(jnp.int32) + PAD_IDX
            cmb = ttb[p][sl] * 32 + ipb[p][sl]
            if 16 * (g + 1) > L:
                valid = lax.iota(jnp.int32, 16) < (L - 16 * g)
                pos = jnp.where(valid, pos, 0)
                cmb = jnp.where(valid, cmb, 0)
            posb[p][sl] = pos
            combb[p][sl] = cmb

    def compute(p):
        acc = accb[p]

        def grp_body(g, carry1):
            tbase = 16 * g
            pv = posb[p][pl.ds(tbase, 16)]
            cv = combb[p][pl.ds(tbase, 16)]
            for k in range(16):
                q = pv[k]
                cmb = cv[k]
                t = tbase + k
                x = []
                for j in range(NH):
                    sj = pl.ds(16 * j, 16)
                    x.append(acc[t, sj] + pe_v[q, sj] + c_v[cmb, sj])
                s = x[0]
                ssq = x[0] * x[0]
                for j in range(1, NH):
                    s = s + x[j]
                    ssq = ssq + x[j] * x[j]
                mean = jnp.sum(s) * (1.0 / HIDDEN)
                var = jnp.sum(ssq) * (1.0 / HIDDEN) - mean * mean + EPS
                vs = _rsqrt_vec(jnp.broadcast_to(var, (16,)))
                meanv = jnp.broadcast_to(mean, (16,))
                for j in range(NH):
                    g2 = vs * gvec[j]
                    b2 = bvec[j] - meanv * g2
                    acc[t, pl.ds(16 * j, 16)] = x[j] * g2 + b2
            return carry1
        lax.fori_loop(0, NG, grp_body, 0)

    def pair_body(k, carry):
        r0 = 2 * k
        r1 = r0 + 1
        hi0 = prefetch_ids(0, r0)
        hi1 = prefetch_ids(1, r1)
        for h in hi0:
            h.wait()
        cumsum_comb(0)
        g0 = start_gather(0)
        for h in hi1:
            h.wait()
        cumsum_comb(1)
        for h in g0:
            h.wait()
        g1 = start_gather(1)
        compute(0)
        o0 = start_out(0, r0)
        for h in g1:
            h.wait()
        compute(1)
        o1 = start_out(1, r1)
        o0.wait()
        o1.wait()
        return carry
    lax.fori_loop(0, rows_per_w // 2, pair_body, 0)


def kernel(input_ids, token_type_ids, item_position_ids, word_embeddings,
           position_embeddings, token_type_embeddings, item_position_embeddings,
           ln_gamma, ln_beta):
    ids = input_ids.astype(jnp.int32)
    tt = token_type_ids.astype(jnp.int32)
    ip = item_position_ids.astype(jnp.int32)

    info = plsc.get_sparse_core_info()
    nc = info.num_cores
    mesh = plsc.VectorSubcoreMesh(core_axis_name="c", subcore_axis_name="s")
    scratch = [
        pltpu.VMEM((POS_ROWS, HIDDEN), jnp.float32),
        pltpu.VMEM((128, HIDDEN), jnp.float32),
        pltpu.VMEM((4, HIDDEN), jnp.float32),
        pltpu.VMEM((32, HIDDEN), jnp.float32),
        pltpu.VMEM((HIDDEN,), jnp.float32),
        pltpu.VMEM((HIDDEN,), jnp.float32),
    ]
    for _ in range(2):
        scratch += [
            pltpu.VMEM((LPAD,), jnp.int32),
            pltpu.VMEM((LPAD,), jnp.int32),
            pltpu.VMEM((LPAD,), jnp.int32),
            pltpu.VMEM((LPAD,), jnp.int32),
            pltpu.VMEM((LPAD,), jnp.int32),
            pltpu.VMEM((LPAD, HIDDEN), jnp.float32),
        ]
    scratch += [pltpu.SemaphoreType.DMA] * 6

    def body(*refs):
        _body(nc, *refs)

    f = pl.kernel(
        body,
        out_type=jax.ShapeDtypeStruct((B, L, HIDDEN), jnp.float32),
        mesh=mesh,
        scratch_types=scratch,
        compiler_params=pltpu.CompilerParams(
            needs_layout_passes=False, use_tc_tiling_on_sc=False),
    )
    return f(ids, tt, ip, word_embeddings, position_embeddings,
             token_type_embeddings, item_position_embeddings,
             ln_gamma, ln_beta)


# speedup vs baseline: 6.5906x; 1.0038x over previous
import jax
import jax.numpy as jnp
from jax import lax
from jax.experimental import pallas as pl
from jax.experimental.pallas import tpu as pltpu
from jax.experimental.pallas import tpu_sc as plsc

VOCAB = 100000
HIDDEN = 128
PAD_IDX = 1
B, L = 4096, 200
EPS = 1e-5
LPAD = 208
NG = LPAD // 16
POS_ROWS = 208
NH = HIDDEN // 16
C0, C1 = 104, 96


def _rsqrt_vec(v):
    i = plsc.bitcast(v, jnp.int32)
    y = plsc.bitcast(jnp.int32(0x5F3759DF) - (i >> 1), jnp.float32)
    for _ in range(3):
        y = y * (1.5 - 0.5 * v * y * y)
    return y


def _body(nc, ids_hbm, tt_hbm, ip_hbm, we_hbm, pe_hbm, tte_hbm, ipe_hbm,
          g_hbm, b_hbm, out_hbm,
          pe_v, c_v, tte_v, ipe_v, g_v, b_v,
          ids0, tt0, ip0, pos0, comb0, acc0,
          ids1, tt1, ip1, pos1, comb1, acc1,
          sem_i0, sem_i1, sem_g0, sem_g1, sem_o0, sem_o1):
    rows_per_w = B // (nc * 16)
    wid = lax.axis_index("s") * nc + lax.axis_index("c")
    base = wid * rows_per_w
    last = rows_per_w - 1

    idsb = (ids0, ids1)
    ttb = (tt0, tt1)
    ipb = (ip0, ip1)
    posb = (pos0, pos1)
    combb = (comb0, comb1)
    accb = (acc0, acc1)
    sem_i = (sem_i0, sem_i1)
    sem_g = (sem_g0, sem_g1)
    sem_o = (sem_o0, sem_o1)

    pltpu.sync_copy(pe_hbm.at[pl.ds(0, POS_ROWS)], pe_v)
    pltpu.sync_copy(tte_hbm, tte_v)
    pltpu.sync_copy(ipe_hbm, ipe_v)
    pltpu.sync_copy(g_hbm, g_v)
    pltpu.sync_copy(b_hbm, b_v)

    @plsc.parallel_loop(0, 128)
    def _build_c(r):
        t = r >> 5
        ii = r & 31
        for j in range(NH):
            sj = pl.ds(16 * j, 16)
            c_v[r, sj] = tte_v[t, sj] + ipe_v[ii, sj]

    gvec = [g_v[pl.ds(16 * j, 16)] for j in range(NH)]
    bvec = [b_v[pl.ds(16 * j, 16)] for j in range(NH)]

    def prefetch_ids(p, r):
        row = base + r
        return (
            pltpu.async_copy(ids_hbm.at[row], idsb[p].at[pl.ds(0, L)], sem_i[p]),
            pltpu.async_copy(tt_hbm.at[row], ttb[p].at[pl.ds(0, L)], sem_i[p]),
            pltpu.async_copy(ip_hbm.at[row], ipb[p].at[pl.ds(0, L)], sem_i[p]),
        )

    def start_gather(p):
        h1 = pltpu.async_copy(we_hbm.at[idsb[p].at[pl.ds(0, C0)]],
                              accb[p].at[pl.ds(0, C0)], sem_g[p])
        h2 = pltpu.async_copy(we_hbm.at[idsb[p].at[pl.ds(C0, C1)]],
                              accb[p].at[pl.ds(C0, C1)], sem_g[p])
        return h1, h2

    def start_out(p, r):
        return pltpu.async_copy(accb[p].at[pl.ds(0, L)], out_hbm.at[base + r],
                                sem_o[p])

    def cumsum_comb(p):
        carry = jnp.float32(0.0)
        for g in range(NG):
            sl = pl.ds(16 * g, 16)
            idsg = idsb[p][sl]
            m = jnp.where(idsg != PAD_IDX, jnp.float32(1.0), jnp.float32(0.0))
            cs = plsc.cumsum(m) + carry
            carry = carry + jnp.sum(m)
            pos = (cs * m).astype(jnp.int32) + PAD_IDX
            cmb = ttb[p][sl] * 32 + ipb[p][sl]
            if 16 * (g + 1) > L:
                valid = lax.iota(jnp.int32, 16) < (L - 16 * g)
                pos = jnp.where(valid, pos, 0)
                cmb = jnp.where(valid, cmb, 0)
            posb[p][sl] = pos
            combb[p][sl] = cmb

    def compute(p):
        acc = accb[p]

        @plsc.parallel_loop(0, NG)
        def grp_body(g):
            tbase = 16 * g
            pv = posb[p][pl.ds(tbase, 16)]
            cv = combb[p][pl.ds(tbase, 16)]
            for k in range(16):
                q = pv[k]
                cmb = cv[k]
                t = tbase + k
                x = []
                for j in range(NH):
                    sj = pl.ds(16 * j, 16)
                    x.append(acc[t, sj] + pe_v[q, sj] + c_v[cmb, sj])
                s = x[0]
                ssq = x[0] * x[0]
                for j in range(1, NH):
                    s = s + x[j]
                    ssq = ssq + x[j] * x[j]
                mean = jnp.sum(s) * (1.0 / HIDDEN)
                var = jnp.sum(ssq) * (1.0 / HIDDEN) - mean * mean + EPS
                vs = _rsqrt_vec(jnp.broadcast_to(var, (16,)))
                meanv = jnp.broadcast_to(mean, (16,))
                for j in range(NH):
                    g2 = vs * gvec[j]
                    b2 = bvec[j] - meanv * g2
                    acc[t, pl.ds(16 * j, 16)] = x[j] * g2 + b2

    def pair_body(k, carry):
        r0 = 2 * k
        r1 = r0 + 1
        hi0 = prefetch_ids(0, r0)
        hi1 = prefetch_ids(1, r1)
        for h in hi0:
            h.wait()
        cumsum_comb(0)
        g0 = start_gather(0)
        for h in hi1:
            h.wait()
        cumsum_comb(1)
        for h in g0:
            h.wait()
        g1 = start_gather(1)
        compute(0)
        o0 = start_out(0, r0)
        for h in g1:
            h.wait()
        compute(1)
        o1 = start_out(1, r1)
        o0.wait()
        o1.wait()
        return carry
    lax.fori_loop(0, rows_per_w // 2, pair_body, 0)


def kernel(input_ids, token_type_ids, item_position_ids, word_embeddings,
           position_embeddings, token_type_embeddings, item_position_embeddings,
           ln_gamma, ln_beta):
    ids = input_ids.astype(jnp.int32)
    tt = token_type_ids.astype(jnp.int32)
    ip = item_position_ids.astype(jnp.int32)

    info = plsc.get_sparse_core_info()
    nc = info.num_cores
    mesh = plsc.VectorSubcoreMesh(core_axis_name="c", subcore_axis_name="s")
    scratch = [
        pltpu.VMEM((POS_ROWS, HIDDEN), jnp.float32),
        pltpu.VMEM((128, HIDDEN), jnp.float32),
        pltpu.VMEM((4, HIDDEN), jnp.float32),
        pltpu.VMEM((32, HIDDEN), jnp.float32),
        pltpu.VMEM((HIDDEN,), jnp.float32),
        pltpu.VMEM((HIDDEN,), jnp.float32),
    ]
    for _ in range(2):
        scratch += [
            pltpu.VMEM((LPAD,), jnp.int32),
            pltpu.VMEM((LPAD,), jnp.int32),
            pltpu.VMEM((LPAD,), jnp.int32),
            pltpu.VMEM((LPAD,), jnp.int32),
            pltpu.VMEM((LPAD,), jnp.int32),
            pltpu.VMEM((LPAD, HIDDEN), jnp.float32),
        ]
    scratch += [pltpu.SemaphoreType.DMA] * 6

    def body(*refs):
        _body(nc, *refs)

    f = pl.kernel(
        body,
        out_type=jax.ShapeDtypeStruct((B, L, HIDDEN), jnp.float32),
        mesh=mesh,
        scratch_types=scratch,
        compiler_params=pltpu.CompilerParams(
            needs_layout_passes=False, use_tc_tiling_on_sc=False),
    )
    return f(ids, tt, ip, word_embeddings, position_embeddings,
             token_type_embeddings, item_position_embeddings,
             ln_gamma, ln_beta)


# speedup vs baseline: 8.9560x; 1.3589x over previous
import jax
import jax.numpy as jnp
from jax import lax
from jax.experimental import pallas as pl
from jax.experimental.pallas import tpu as pltpu
from jax.experimental.pallas import tpu_sc as plsc

VOCAB = 100000
HIDDEN = 128
PAD_IDX = 1
B, L = 4096, 200
EPS = 1e-5
LPAD = 208
NG = LPAD // 16
POS_ROWS = 208
NH = HIDDEN // 16
C0, C1 = 104, 96
Q_OFF = 256
M_OFF = 512
V_OFF = 528
RED_STRIDE = 576


def _rsqrt_vec(v):
    i = plsc.bitcast(v, jnp.int32)
    y = plsc.bitcast(jnp.int32(0x5F3759DF) - (i >> 1), jnp.float32)
    for _ in range(3):
        y = y * (1.5 - 0.5 * v * y * y)
    return y


def _body(nc, ids_hbm, tt_hbm, ip_hbm, we_hbm, pe_hbm, tte_hbm, ipe_hbm,
          g_hbm, b_hbm, out_hbm,
          pe_v, c_v, tte_v, ipe_v, g_v, b_v,
          ids0, tt0, ip0, pos0, comb0, acc0, red0,
          ids1, tt1, ip1, pos1, comb1, acc1, red1,
          sem_i0, sem_i1, sem_g0, sem_g1, sem_o0, sem_o1):
    rows_per_w = B // (nc * 16)
    wid = lax.axis_index("s") * nc + lax.axis_index("c")
    base = wid * rows_per_w
    last = rows_per_w - 1

    idsb = (ids0, ids1)
    ttb = (tt0, tt1)
    ipb = (ip0, ip1)
    posb = (pos0, pos1)
    combb = (comb0, comb1)
    accb = (acc0, acc1)
    redb = (red0, red1)
    sem_i = (sem_i0, sem_i1)
    sem_g = (sem_g0, sem_g1)
    sem_o = (sem_o0, sem_o1)

    pltpu.sync_copy(pe_hbm.at[pl.ds(0, POS_ROWS)], pe_v)
    pltpu.sync_copy(tte_hbm, tte_v)
    pltpu.sync_copy(ipe_hbm, ipe_v)
    pltpu.sync_copy(g_hbm, g_v)
    pltpu.sync_copy(b_hbm, b_v)

    @plsc.parallel_loop(0, 128)
    def _build_c(r):
        t = r >> 5
        ii = r & 31
        for j in range(NH):
            sj = pl.ds(16 * j, 16)
            c_v[r, sj] = tte_v[t, sj] + ipe_v[ii, sj]

    gvec = [g_v[pl.ds(16 * j, 16)] for j in range(NH)]
    bvec = [b_v[pl.ds(16 * j, 16)] for j in range(NH)]

    def prefetch_ids(p, r):
        row = base + r
        return (
            pltpu.async_copy(ids_hbm.at[row], idsb[p].at[pl.ds(0, L)], sem_i[p]),
            pltpu.async_copy(tt_hbm.at[row], ttb[p].at[pl.ds(0, L)], sem_i[p]),
            pltpu.async_copy(ip_hbm.at[row], ipb[p].at[pl.ds(0, L)], sem_i[p]),
        )

    def start_gather(p):
        h1 = pltpu.async_copy(we_hbm.at[idsb[p].at[pl.ds(0, C0)]],
                              accb[p].at[pl.ds(0, C0)], sem_g[p])
        h2 = pltpu.async_copy(we_hbm.at[idsb[p].at[pl.ds(C0, C1)]],
                              accb[p].at[pl.ds(C0, C1)], sem_g[p])
        return h1, h2

    def start_out(p, r):
        return pltpu.async_copy(accb[p].at[pl.ds(0, L)], out_hbm.at[base + r],
                                sem_o[p])

    def cumsum_comb(p):
        carry = jnp.float32(0.0)
        for g in range(NG):
            sl = pl.ds(16 * g, 16)
            idsg = idsb[p][sl]
            m = jnp.where(idsg != PAD_IDX, jnp.float32(1.0), jnp.float32(0.0))
            cs = plsc.cumsum(m) + carry
            carry = carry + jnp.sum(m)
            pos = (cs * m).astype(jnp.int32) + PAD_IDX
            cmb = ttb[p][sl] * 32 + ipb[p][sl]
            if 16 * (g + 1) > L:
                valid = lax.iota(jnp.int32, 16) < (L - 16 * g)
                pos = jnp.where(valid, pos, 0)
                cmb = jnp.where(valid, cmb, 0)
            posb[p][sl] = pos
            combb[p][sl] = cmb

    lane16 = lax.iota(jnp.int32, 16) * 16
    cidx = [lane16 + c for c in range(16)]

    def compute(p):
        acc = accb[p]
        red = redb[p]

        @plsc.parallel_loop(0, NG)
        def grp_body(g):
            tbase = 16 * g
            rbase = g * RED_STRIDE
            pv = posb[p][pl.ds(tbase, 16)]
            cv = combb[p][pl.ds(tbase, 16)]
            for k in range(16):
                q = pv[k]
                cmb = cv[k]
                t = tbase + k
                x = []
                for j in range(NH):
                    sj = pl.ds(16 * j, 16)
                    x.append(acc[t, sj] + pe_v[q, sj] + c_v[cmb, sj])
                s = x[0]
                ssq = x[0] * x[0]
                for j in range(1, NH):
                    s = s + x[j]
                    ssq = ssq + x[j] * x[j]
                for j in range(NH):
                    acc[t, pl.ds(16 * j, 16)] = x[j]
                red[pl.ds(rbase + 16 * k, 16)] = s
                red[pl.ds(rbase + Q_OFF + 16 * k, 16)] = ssq
            basev = jnp.broadcast_to(rbase, (16,))
            tot = plsc.load_gather(red, [basev + cidx[0]])
            totq = plsc.load_gather(red, [basev + (cidx[0] + Q_OFF)])
            for c in range(1, 16):
                tot = tot + plsc.load_gather(red, [basev + cidx[c]])
                totq = totq + plsc.load_gather(red, [basev + (cidx[c] + Q_OFF)])
            meanv = tot * (1.0 / HIDDEN)
            varv = totq * (1.0 / HIDDEN) - meanv * meanv + EPS
            vsv = _rsqrt_vec(varv)
            red[pl.ds(rbase + M_OFF, 16)] = meanv
            red[pl.ds(rbase + V_OFF, 16)] = vsv
            for k in range(16):
                t = tbase + k
                mv = plsc.load_gather(
                    red, [jnp.broadcast_to(rbase + (M_OFF + k), (16,))])
                vv = plsc.load_gather(
                    red, [jnp.broadcast_to(rbase + (V_OFF + k), (16,))])
                for j in range(NH):
                    sj = pl.ds(16 * j, 16)
                    acc[t, sj] = (acc[t, sj] - mv) * vv * gvec[j] + bvec[j]

    def pair_body(k, carry):
        r0 = 2 * k
        r1 = r0 + 1
        hi0 = prefetch_ids(0, r0)
        hi1 = prefetch_ids(1, r1)
        for h in hi0:
            h.wait()
        cumsum_comb(0)
        g0 = start_gather(0)
        for h in hi1:
            h.wait()
        cumsum_comb(1)
        for h in g0:
            h.wait()
        g1 = start_gather(1)
        compute(0)
        o0 = start_out(0, r0)
        for h in g1:
            h.wait()
        compute(1)
        o1 = start_out(1, r1)
        o0.wait()
        o1.wait()
        return carry
    lax.fori_loop(0, rows_per_w // 2, pair_body, 0)


def kernel(input_ids, token_type_ids, item_position_ids, word_embeddings,
           position_embeddings, token_type_embeddings, item_position_embeddings,
           ln_gamma, ln_beta):
    ids = input_ids.astype(jnp.int32)
    tt = token_type_ids.astype(jnp.int32)
    ip = item_position_ids.astype(jnp.int32)

    info = plsc.get_sparse_core_info()
    nc = info.num_cores
    mesh = plsc.VectorSubcoreMesh(core_axis_name="c", subcore_axis_name="s")
    scratch = [
        pltpu.VMEM((POS_ROWS, HIDDEN), jnp.float32),
        pltpu.VMEM((128, HIDDEN), jnp.float32),
        pltpu.VMEM((4, HIDDEN), jnp.float32),
        pltpu.VMEM((32, HIDDEN), jnp.float32),
        pltpu.VMEM((HIDDEN,), jnp.float32),
        pltpu.VMEM((HIDDEN,), jnp.float32),
    ]
    for _ in range(2):
        scratch += [
            pltpu.VMEM((LPAD,), jnp.int32),
            pltpu.VMEM((LPAD,), jnp.int32),
            pltpu.VMEM((LPAD,), jnp.int32),
            pltpu.VMEM((LPAD,), jnp.int32),
            pltpu.VMEM((LPAD,), jnp.int32),
            pltpu.VMEM((LPAD, HIDDEN), jnp.float32),
            pltpu.VMEM((NG * RED_STRIDE,), jnp.float32),
        ]
    scratch += [pltpu.SemaphoreType.DMA] * 6

    def body(*refs):
        _body(nc, *refs)

    f = pl.kernel(
        body,
        out_type=jax.ShapeDtypeStruct((B, L, HIDDEN), jnp.float32),
        mesh=mesh,
        scratch_types=scratch,
        compiler_params=pltpu.CompilerParams(
            needs_layout_passes=False, use_tc_tiling_on_sc=False),
    )
    return f(ids, tt, ip, word_embeddings, position_embeddings,
             token_type_embeddings, item_position_embeddings,
             ln_gamma, ln_beta)
